# baseline (speedup 1.0000x reference)
import jax
import jax.numpy as jnp
from jax.experimental import pallas as pl
from jax.experimental.pallas import tpu as pltpu

_B = 1024
_L = 200
_N = 1000
_BM = 40


def _infectivity_body(ti_t_ref, tjs_t_ref, hist_ref, emb_ref, out_ref,
                      gt_scratch):
    @pl.when(pl.program_id(0) == 0)
    def _():
        gt_scratch[...] = jnp.exp(tjs_t_ref[...] - ti_t_ref[...])

    hist = hist_ref[...].astype(jnp.float32)
    a = jax.lax.dot_general(
        emb_ref[...], hist, (((1,), (1,)), ((), ())),
        preferred_element_type=jnp.float32)
    out_ref[...] = jax.lax.dot_general(
        a, gt_scratch[...], (((1,), (0,)), ((), ())),
        preferred_element_type=jnp.float32)


def kernel(ti, tjs, ci, cjs, emb_weight):
    del ci
    ti_t = ti.reshape(1, _B)
    tjs_t = tjs.reshape(_L, 1)
    hist = cjs.reshape(_L, _N)
    out2d = pl.pallas_call(
        _infectivity_body,
        grid=(_N // _BM,),
        in_specs=[
            pl.BlockSpec((1, _B), lambda i: (0, 0)),
            pl.BlockSpec((_L, 1), lambda i: (0, 0)),
            pl.BlockSpec((_L, _N), lambda i: (0, 0)),
            pl.BlockSpec((_BM, _N), lambda i: (i, 0)),
        ],
        out_specs=pl.BlockSpec((_BM, _B), lambda i: (i, 0)),
        out_shape=jax.ShapeDtypeStruct((_N, _B), jnp.float32),
        scratch_shapes=[pltpu.VMEM((_L, _B), jnp.float32)],
    )(ti_t, tjs_t, hist, emb_weight)
    return out2d[:, :, None]


# speedup vs baseline: 1.8702x; 1.8702x over previous
import jax
import jax.numpy as jnp
from jax.experimental import pallas as pl
from jax.experimental.pallas import tpu as pltpu

_B = 1024
_L = 200
_N = 1000
def _infectivity_body(ti_t_ref, tjs_t_ref, hist_ref, emb_ref, out_ref):
    gt_t = jnp.exp(tjs_t_ref[...] - ti_t_ref[...])
    hist = hist_ref[...].astype(jnp.float32)
    a = jax.lax.dot_general(
        emb_ref[...], hist, (((1,), (1,)), ((), ())),
        preferred_element_type=jnp.float32)
    out_ref[...] = jax.lax.dot_general(
        a, gt_t, (((1,), (0,)), ((), ())),
        preferred_element_type=jnp.float32)


def kernel(ti, tjs, ci, cjs, emb_weight):
    del ci
    ti_t = ti.reshape(1, _B)
    tjs_t = tjs.reshape(_L, 1)
    hist = cjs.reshape(_L, _N)
    out2d = pl.pallas_call(
        _infectivity_body,
        out_shape=jax.ShapeDtypeStruct((_N, _B), jnp.float32),
    )(ti_t, tjs_t, hist, emb_weight)
    return out2d[:, :, None]
